# Initial kernel scaffold; baseline (speedup 1.0000x reference)
import jax
import jax.numpy as jnp
from jax.experimental import pallas as pl


def kernel(X, W1_mu, W1_logsig2, W2_mu, W2_logsig2, X_idx):
    raise NotImplementedError("write your pallas kernel here")


# speedup vs baseline: 3.8234x; 3.8234x over previous
import functools

import jax
import jax.numpy as jnp
from jax import lax
from jax.experimental import pallas as pl
from jax.experimental.pallas import tpu as pltpu
from jax.experimental.pallas import tpu_sc as plsc

EPS = 1e-8


def _vb_body(x_ref, w1mu_ref, w1ls_ref, w2mu_ref, w2ls_ref, out_ref):
    x = x_ref[...]
    w1mu = w1mu_ref[...]
    sig21 = jnp.exp(w1ls_ref[...])
    w2mu = w2mu_ref[...]
    sig22 = jnp.exp(w2ls_ref[...])

    m1 = jnp.dot(x, w1mu, preferred_element_type=jnp.float32)
    v1 = jnp.dot(x * x, sig21, preferred_element_type=jnp.float32)
    scale = (2.0 / w1mu.shape[1]) ** 0.5
    m1 = scale * jnp.maximum(m1, 0.0)
    v1 = (scale * scale) * v1

    m2 = jnp.dot(m1, w2mu, preferred_element_type=jnp.float32)
    v2 = (jnp.dot(m1 * m1 + v1, sig22, preferred_element_type=jnp.float32)
          + jnp.dot(v1, w2mu * w2mu, preferred_element_type=jnp.float32))

    p = 1.0 / (v2 + EPS)
    out_ref[:, : p.shape[1]] = p
    out_ref[:, p.shape[1]:] = p * m2


def _vb_layers(X, W1_mu, W1_logsig2, W2_mu, W2_logsig2, block_rows):
    n, d0 = X.shape
    d1 = W1_mu.shape[1]
    d2 = W2_mu.shape[1]
    grid = n // block_rows
    return pl.pallas_call(
        _vb_body,
        grid=(grid,),
        in_specs=[
            pl.BlockSpec((block_rows, d0), lambda i: (i, 0)),
            pl.BlockSpec((d0, d1), lambda i: (0, 0)),
            pl.BlockSpec((d0, d1), lambda i: (0, 0)),
            pl.BlockSpec((d1, d2), lambda i: (0, 0)),
            pl.BlockSpec((d1, d2), lambda i: (0, 0)),
        ],
        out_specs=pl.BlockSpec((block_rows, 2 * d2), lambda i: (i, 0)),
        out_shape=jax.ShapeDtypeStruct((n, 2 * d2), jnp.float32),
    )(X, W1_mu, W1_logsig2, W2_mu, W2_logsig2)


def _make_seg_sum(n, num_seg_pad, width, chunk):
    info = plsc.get_sparse_core_info()
    nc, ns = info.num_cores, info.num_subcores
    nw = nc * ns
    rows_per_tile = n // nw
    n_chunks = rows_per_tile // chunk
    segs_per_tile = num_seg_pad // ns

    mesh = plsc.VectorSubcoreMesh(core_axis_name="c", subcore_axis_name="s")

    @functools.partial(
        pl.kernel,
        out_type=jax.ShapeDtypeStruct((nc, num_seg_pad, width), jnp.float32),
        mesh=mesh,
        scratch_types=[
            pltpu.VMEM((chunk,), jnp.int32),
            pltpu.VMEM((chunk, width), jnp.float32),
            pltpu.VMEM_SHARED((num_seg_pad, width), jnp.float32),
        ],
    )
    def seg_sum(pw_hbm, idx_hbm, zeros_hbm, part_hbm, idx_v, rows_v, acc_sh):
        cid = lax.axis_index("c")
        sid = lax.axis_index("s")
        wid = sid * nc + cid

        pltpu.sync_copy(
            zeros_hbm.at[pl.ds(sid * segs_per_tile, segs_per_tile)],
            acc_sh.at[pl.ds(sid * segs_per_tile, segs_per_tile)],
        )
        plsc.subcore_barrier()

        base = wid * rows_per_tile

        def body(i, carry):
            off = base + i * chunk
            pltpu.sync_copy(idx_hbm.at[pl.ds(off, chunk)], idx_v)
            pltpu.sync_copy(pw_hbm.at[pl.ds(off, chunk)], rows_v)
            pltpu.sync_copy(rows_v, acc_sh.at[idx_v], add=True)
            return carry

        lax.fori_loop(0, n_chunks, body, 0)
        plsc.subcore_barrier()

        pltpu.sync_copy(
            acc_sh.at[pl.ds(sid * segs_per_tile, segs_per_tile)],
            part_hbm.at[cid, pl.ds(sid * segs_per_tile, segs_per_tile)],
        )

    return seg_sum


def _fin_body(part_ref, means_ref, vars_ref):
    num_seg, d2 = means_ref.shape
    s = part_ref[0, :num_seg, :] + part_ref[1, :num_seg, :]
    w = s[:, :d2] + EPS
    var = 1.0 / w
    means_ref[...] = s[:, d2:] * var
    vars_ref[...] = var


def _finalize(part, num_seg, d2):
    return pl.pallas_call(
        _fin_body,
        out_shape=(
            jax.ShapeDtypeStruct((num_seg, d2), jnp.float32),
            jax.ShapeDtypeStruct((num_seg, d2), jnp.float32),
        ),
    )(part)


def kernel(X, W1_mu, W1_logsig2, W2_mu, W2_logsig2, X_idx):
    n = X.shape[0]
    d2 = W2_mu.shape[1]
    num_seg = 10000
    num_seg_pad = 10240
    width = 2 * d2

    pw = _vb_layers(X, W1_mu, W1_logsig2, W2_mu, W2_logsig2, block_rows=2000)
    zeros = jnp.zeros((num_seg_pad, width), jnp.float32)
    part = _make_seg_sum(n, num_seg_pad, width, chunk=80)(pw, X_idx, zeros)
    means, vars_ = _finalize(part, num_seg, d2)
    return means, vars_


# speedup vs baseline: 4.8783x; 1.2759x over previous
import functools

import jax
import jax.numpy as jnp
from jax import lax
from jax.experimental import pallas as pl
from jax.experimental.pallas import tpu as pltpu
from jax.experimental.pallas import tpu_sc as plsc

EPS = 1e-8


def _vb_body(x_ref, w1mu_ref, w1ls_ref, w2mu_ref, w2ls_ref, out_ref):
    x = x_ref[...]
    w1mu = w1mu_ref[...]
    sig21 = jnp.exp(w1ls_ref[...])
    w2mu = w2mu_ref[...]
    sig22 = jnp.exp(w2ls_ref[...])

    m1 = jnp.dot(x, w1mu, preferred_element_type=jnp.float32)
    v1 = jnp.dot(x * x, sig21, preferred_element_type=jnp.float32)
    scale = (2.0 / w1mu.shape[1]) ** 0.5
    m1 = scale * jnp.maximum(m1, 0.0)
    v1 = (scale * scale) * v1

    m2 = jnp.dot(m1, w2mu, preferred_element_type=jnp.float32)
    v2 = (jnp.dot(m1 * m1 + v1, sig22, preferred_element_type=jnp.float32)
          + jnp.dot(v1, w2mu * w2mu, preferred_element_type=jnp.float32))

    p = 1.0 / (v2 + EPS)
    out_ref[:, : p.shape[1]] = p
    out_ref[:, p.shape[1]:] = p * m2


def _vb_layers(X, W1_mu, W1_logsig2, W2_mu, W2_logsig2, block_rows):
    n, d0 = X.shape
    d1 = W1_mu.shape[1]
    d2 = W2_mu.shape[1]
    grid = n // block_rows
    return pl.pallas_call(
        _vb_body,
        grid=(grid,),
        in_specs=[
            pl.BlockSpec((block_rows, d0), lambda i: (i, 0)),
            pl.BlockSpec((d0, d1), lambda i: (0, 0)),
            pl.BlockSpec((d0, d1), lambda i: (0, 0)),
            pl.BlockSpec((d1, d2), lambda i: (0, 0)),
            pl.BlockSpec((d1, d2), lambda i: (0, 0)),
        ],
        out_specs=pl.BlockSpec((block_rows, 2 * d2), lambda i: (i, 0)),
        out_shape=jax.ShapeDtypeStruct((n, 2 * d2), jnp.float32),
    )(X, W1_mu, W1_logsig2, W2_mu, W2_logsig2)


def _make_seg_sum(n, num_seg_pad, width, chunk):
    info = plsc.get_sparse_core_info()
    nc, ns = info.num_cores, info.num_subcores
    nw = nc * ns
    rows_per_tile = n // nw
    n_chunks = rows_per_tile // chunk
    segs_per_tile = num_seg_pad // ns

    mesh = plsc.VectorSubcoreMesh(core_axis_name="c", subcore_axis_name="s")

    @functools.partial(
        pl.kernel,
        out_type=jax.ShapeDtypeStruct((nc, num_seg_pad, width), jnp.float32),
        mesh=mesh,
        scratch_types=[
            pltpu.VMEM((n_chunks, chunk), jnp.int32),
            pltpu.VMEM((chunk, width), jnp.float32),
            pltpu.VMEM((chunk, width), jnp.float32),
            pltpu.VMEM_SHARED((num_seg_pad, width), jnp.float32),
            pltpu.SemaphoreType.DMA,
            pltpu.SemaphoreType.DMA,
        ],
    )
    def seg_sum(pw_hbm, idx_hbm, zeros_hbm, part_hbm,
                idx_v, buf0, buf1, acc_sh, sem0, sem1):
        cid = lax.axis_index("c")
        sid = lax.axis_index("s")
        wid = sid * nc + cid
        base = wid * rows_per_tile
        bufs = (buf0, buf1)
        sems = (sem0, sem1)

        pltpu.sync_copy(idx_hbm.at[wid], idx_v)
        pltpu.sync_copy(
            zeros_hbm.at[pl.ds(sid * segs_per_tile, segs_per_tile)],
            acc_sh.at[pl.ds(sid * segs_per_tile, segs_per_tile)],
        )
        plsc.subcore_barrier()

        def issue(g, b):
            pltpu.async_copy(pw_hbm.at[pl.ds(base + g * chunk, chunk)],
                             bufs[b], sems[b])

        for b in range(2):
            issue(b, b)

        def body(j, carry):
            for b in range(2):
                g = 2 * j + b
                pltpu.make_async_copy(pw_hbm.at[pl.ds(0, chunk)],
                                      bufs[b], sems[b]).wait()
                pltpu.sync_copy(bufs[b], acc_sh.at[idx_v.at[g]], add=True)
                issue(jnp.minimum(g + 2, n_chunks - 1), b)
            return carry

        lax.fori_loop(0, (n_chunks - 1) // 2, body, 0)

        gl = n_chunks - 1
        pltpu.make_async_copy(pw_hbm.at[pl.ds(0, chunk)], bufs[0], sems[0]).wait()
        pltpu.sync_copy(bufs[0], acc_sh.at[idx_v.at[gl]], add=True)
        pltpu.make_async_copy(pw_hbm.at[pl.ds(0, chunk)], bufs[1], sems[1]).wait()
        plsc.subcore_barrier()

        pltpu.sync_copy(
            acc_sh.at[pl.ds(sid * segs_per_tile, segs_per_tile)],
            part_hbm.at[cid, pl.ds(sid * segs_per_tile, segs_per_tile)],
        )

    return seg_sum


def _fin_body(part_ref, means_ref, vars_ref):
    num_seg, d2 = means_ref.shape
    s = part_ref[0, :num_seg, :] + part_ref[1, :num_seg, :]
    w = s[:, :d2] + EPS
    var = 1.0 / w
    means_ref[...] = s[:, d2:] * var
    vars_ref[...] = var


def _finalize(part, num_seg, d2):
    return pl.pallas_call(
        _fin_body,
        out_shape=(
            jax.ShapeDtypeStruct((num_seg, d2), jnp.float32),
            jax.ShapeDtypeStruct((num_seg, d2), jnp.float32),
        ),
    )(part)


def kernel(X, W1_mu, W1_logsig2, W2_mu, W2_logsig2, X_idx):
    n = X.shape[0]
    d2 = W2_mu.shape[1]
    num_seg = 10000
    num_seg_pad = 10240
    width = 2 * d2

    chunk = 80
    info = plsc.get_sparse_core_info()
    nw = info.num_cores * info.num_subcores
    n_chunks = n // (nw * chunk)
    assert n_chunks % 2 == 1
    pw = _vb_layers(X, W1_mu, W1_logsig2, W2_mu, W2_logsig2, block_rows=2000)
    zeros = jnp.zeros((num_seg_pad, width), jnp.float32)
    idx3d = X_idx.reshape(nw, n_chunks, chunk)
    part = _make_seg_sum(n, num_seg_pad, width, chunk=chunk)(pw, idx3d, zeros)
    means, vars_ = _finalize(part, num_seg, d2)
    return means, vars_
